# baseline (speedup 1.0000x reference)
import jax
import jax.numpy as jnp
from jax.experimental import pallas as pl
from jax.experimental.pallas import tpu as pltpu

_B = 8
_N = 2048
_CH = 256
_PER_B = _N // _CH
_CHUNKS = _B * _PER_B
_NBUF = 12


def _body(emb_ref, ns_ref, cov_hbm, out_hbm, buf, in_sems, out_sems):
    def in_copy(k):
        b, r0 = k // _PER_B, (k % _PER_B) * _CH
        return pltpu.make_async_copy(
            cov_hbm.at[b, pl.ds(r0, _CH)], buf.at[k % _NBUF], in_sems.at[k % _NBUF]
        )

    def out_copy(k):
        b, r0 = k // _PER_B, (k % _PER_B) * _CH
        return pltpu.make_async_copy(
            buf.at[k % _NBUF], out_hbm.at[b, pl.ds(r0, _CH)], out_sems.at[k % _NBUF]
        )

    row = jax.lax.broadcasted_iota(jnp.int32, (_CH, _CH), 0)
    col = jax.lax.broadcasted_iota(jnp.int32, (_CH, _CH), 1)
    ns = jnp.exp(ns_ref[0, 0])

    for j in range(min(_NBUF, _CHUNKS)):
        in_copy(j).start()

    waited_out = set()
    for k in range(_CHUNKS):
        b, r0 = k // _PER_B, (k % _PER_B) * _CH
        in_copy(k).wait()
        ev = jnp.exp(emb_ref[b, :, pl.ds(r0, _CH)]) + ns
        i = k % _NBUF
        buf[i, :, pl.ds(r0, _CH)] = buf[i, :, pl.ds(r0, _CH)] + jnp.where(
            row == col, ev, 0.0
        )
        out_copy(k).start()
        j = k - 2
        if j >= 0 and j + _NBUF < _CHUNKS:
            out_copy(j).wait()
            waited_out.add(j)
            in_copy(j + _NBUF).start()
    for k in range(_CHUNKS):
        if k not in waited_out:
            out_copy(k).wait()


def kernel(cov, embeddings, noise_scale):
    emb = embeddings[:, :, -1].reshape(_B, 1, _N)
    ns = noise_scale.reshape(1, 1)
    return pl.pallas_call(
        _body,
        in_specs=[
            pl.BlockSpec(memory_space=pltpu.MemorySpace.VMEM),
            pl.BlockSpec(memory_space=pltpu.MemorySpace.VMEM),
            pl.BlockSpec(memory_space=pl.ANY),
        ],
        out_specs=pl.BlockSpec(memory_space=pl.ANY),
        out_shape=jax.ShapeDtypeStruct((_B, _N, _N), jnp.float32),
        scratch_shapes=[
            pltpu.VMEM((_NBUF, _CH, _N), jnp.float32),
            pltpu.SemaphoreType.DMA((_NBUF,)),
            pltpu.SemaphoreType.DMA((_NBUF,)),
        ],
    )(emb, ns, cov)


# speedup vs baseline: 1.0038x; 1.0038x over previous
import jax
import jax.numpy as jnp
from jax.experimental import pallas as pl
from jax.experimental.pallas import tpu as pltpu

_B = 8
_N = 2048
_CH = 1024
_PER_B = _N // _CH
_CHUNKS = _B * _PER_B
_NBUF = 4


def _body(emb_ref, ns_ref, cov_hbm, out_hbm, buf, in_sems, out_sems):
    def in_copy(k):
        b, r0 = k // _PER_B, (k % _PER_B) * _CH
        return pltpu.make_async_copy(
            cov_hbm.at[b, pl.ds(r0, _CH)], buf.at[k % _NBUF], in_sems.at[k % _NBUF]
        )

    def out_copy(k):
        b, r0 = k // _PER_B, (k % _PER_B) * _CH
        return pltpu.make_async_copy(
            buf.at[k % _NBUF], out_hbm.at[b, pl.ds(r0, _CH)], out_sems.at[k % _NBUF]
        )

    row = jax.lax.broadcasted_iota(jnp.int32, (_CH, _CH), 0)
    col = jax.lax.broadcasted_iota(jnp.int32, (_CH, _CH), 1)
    ns = jnp.exp(ns_ref[0, 0])

    for j in range(min(_NBUF, _CHUNKS)):
        in_copy(j).start()

    waited_out = set()
    for k in range(_CHUNKS):
        b, r0 = k // _PER_B, (k % _PER_B) * _CH
        in_copy(k).wait()
        ev = jnp.exp(emb_ref[b, :, pl.ds(r0, _CH)]) + ns
        i = k % _NBUF
        buf[i, :, pl.ds(r0, _CH)] = buf[i, :, pl.ds(r0, _CH)] + jnp.where(
            row == col, ev, 0.0
        )
        out_copy(k).start()
        j = k - 2
        if j >= 0 and j + _NBUF < _CHUNKS:
            out_copy(j).wait()
            waited_out.add(j)
            in_copy(j + _NBUF).start()
    for k in range(_CHUNKS):
        if k not in waited_out:
            out_copy(k).wait()


def kernel(cov, embeddings, noise_scale):
    emb = embeddings[:, :, -1].reshape(_B, 1, _N)
    ns = noise_scale.reshape(1, 1)
    return pl.pallas_call(
        _body,
        in_specs=[
            pl.BlockSpec(memory_space=pltpu.MemorySpace.VMEM),
            pl.BlockSpec(memory_space=pltpu.MemorySpace.VMEM),
            pl.BlockSpec(memory_space=pl.ANY),
        ],
        out_specs=pl.BlockSpec(memory_space=pl.ANY),
        out_shape=jax.ShapeDtypeStruct((_B, _N, _N), jnp.float32),
        scratch_shapes=[
            pltpu.VMEM((_NBUF, _CH, _N), jnp.float32),
            pltpu.SemaphoreType.DMA((_NBUF,)),
            pltpu.SemaphoreType.DMA((_NBUF,)),
        ],
    )(emb, ns, cov)


# speedup vs baseline: 1.0109x; 1.0071x over previous
import jax
import jax.numpy as jnp
from jax.experimental import pallas as pl

_B = 8
_N = 2048
_ROWS = 1024


def _stripe_kernel(emb_ref, ns_ref, cov_ref, out_ref):
    i = pl.program_id(1)
    out_ref[0] = cov_ref[0]
    ev = jnp.exp(emb_ref[0, :, pl.ds(i * _ROWS, _ROWS)]) + jnp.exp(ns_ref[0, 0])
    row = jax.lax.broadcasted_iota(jnp.int32, (_ROWS, _ROWS), 0)
    col = jax.lax.broadcasted_iota(jnp.int32, (_ROWS, _ROWS), 1)
    sub = out_ref[0, :, pl.ds(i * _ROWS, _ROWS)]
    out_ref[0, :, pl.ds(i * _ROWS, _ROWS)] = sub + jnp.where(row == col, ev, 0.0)


def kernel(cov, embeddings, noise_scale):
    emb = embeddings[:, :, -1].reshape(_B, 1, _N)
    ns = noise_scale.reshape(1, 1)
    return pl.pallas_call(
        _stripe_kernel,
        grid=(_B, _N // _ROWS),
        in_specs=[
            pl.BlockSpec((1, 1, _N), lambda b, i: (b, 0, 0)),
            pl.BlockSpec((1, 1), lambda b, i: (0, 0)),
            pl.BlockSpec((1, _ROWS, _N), lambda b, i: (b, i, 0)),
        ],
        out_specs=pl.BlockSpec((1, _ROWS, _N), lambda b, i: (b, i, 0)),
        out_shape=jax.ShapeDtypeStruct((_B, _N, _N), jnp.float32),
    )(emb, ns, cov)
